# initial kernel scaffold (unmeasured)
import jax
import jax.numpy as jnp
from jax import lax
from jax.experimental import pallas as pl
from jax.experimental.pallas import tpu as pltpu

N_DEV = 4
M_CHUNK = 2048
K = 2048
N = 4096
NH = N // 2
MB = 512
NSUB = M_CHUNK // MB
N_SLOTS = 3


def kernel(x, w_mat):
    assert x.shape == (N_DEV * M_CHUNK, K), x.shape
    assert w_mat.shape == (K, N), w_mat.shape

    def body(x_hbm, w_ref, out_hbm, comm, xbuf, stage,
             send_sems, recv_sems, copy_sem, out_sem):
        my = lax.axis_index("i")
        right = lax.rem(my + 1, N_DEV)
        left = lax.rem(my + N_DEV - 1, N_DEV)

        barrier_sem = pltpu.get_barrier_semaphore()
        for nbr in (left, right):
            pl.semaphore_signal(barrier_sem, inc=1, device_id=(nbr,),
                                device_id_type=pl.DeviceIdType.MESH)
        pl.semaphore_wait(barrier_sem, 2)

        def load_x(chunk_j, b):
            cp = pltpu.make_async_copy(
                x_hbm.at[pl.ds(chunk_j * M_CHUNK + b * MB, MB), :], xbuf,
                copy_sem)
            cp.start()
            cp.wait()

        hop = 0
        for r in range(2):
            ncol = r * NH
            for h in range(3):
                send_slot = hop % N_SLOTS
                recv_slot = (hop + 1) % N_SLOTS
                j = lax.rem(my - 1 - h + 2 * N_DEV, N_DEV)
                for b in range(NSUB):
                    load_x(j, b)
                    part = jnp.dot(xbuf[:, :], w_ref[:, ncol:ncol + NH],
                                   preferred_element_type=jnp.float32)
                    rows = pl.ds(b * MB, MB)
                    if h == 0:
                        comm[send_slot, rows, :] = part.astype(jnp.bfloat16)
                    else:
                        acc = part + comm[send_slot, rows, :].astype(jnp.float32)
                        comm[send_slot, rows, :] = acc.astype(jnp.bfloat16)
                rdma = pltpu.make_async_remote_copy(
                    src_ref=comm.at[send_slot],
                    dst_ref=comm.at[recv_slot],
                    send_sem=send_sems.at[send_slot],
                    recv_sem=recv_sems.at[recv_slot],
                    device_id=(right,),
                    device_id_type=pl.DeviceIdType.MESH)
                rdma.start()
                rdma.wait()
                hop += 1
            fin_slot = hop % N_SLOTS
            for b in range(NSUB):
                load_x(my, b)
                part = jnp.dot(xbuf[:, :], w_ref[:, ncol:ncol + NH],
                               preferred_element_type=jnp.float32)
                rows = pl.ds(b * MB, MB)
                stage[:, :] = part + comm[fin_slot, rows, :].astype(jnp.float32)
                ocp = pltpu.make_async_copy(
                    stage, out_hbm.at[rows, pl.ds(ncol, NH)], out_sem)
                ocp.start()
                ocp.wait()

    return pl.pallas_call(
        body,
        out_shape=jax.ShapeDtypeStruct((M_CHUNK, N), jnp.float32),
        in_specs=[
            pl.BlockSpec(memory_space=pltpu.ANY),
            pl.BlockSpec(memory_space=pltpu.VMEM),
        ],
        out_specs=pl.BlockSpec(memory_space=pltpu.ANY),
        scratch_shapes=[
            pltpu.VMEM((N_SLOTS, M_CHUNK, NH), jnp.bfloat16),
            pltpu.VMEM((MB, K), jnp.bfloat16),
            pltpu.VMEM((MB, NH), jnp.float32),
            pltpu.SemaphoreType.DMA((N_SLOTS,)),
            pltpu.SemaphoreType.DMA((N_SLOTS,)),
            pltpu.SemaphoreType.DMA,
            pltpu.SemaphoreType.DMA,
        ],
        compiler_params=pltpu.CompilerParams(collective_id=0),
    )(x, w_mat)


# baseline (device time: 834923 ns/iter reference)
import jax
import jax.numpy as jnp
from jax import lax
from jax.experimental import pallas as pl
from jax.experimental.pallas import tpu as pltpu

N_DEV = 4
M_CHUNK = 2048
K = 2048
N = 4096
NH = N // 2
MB = 512
NSUB = M_CHUNK // MB
WB = 256
N_SLOTS = 2


def kernel(x, w_mat):
    assert x.shape == (N_DEV * M_CHUNK, K), x.shape
    assert w_mat.shape == (K, N), w_mat.shape

    def body(x_hbm, w_hbm, out_hbm, comm, wbuf, wstage, xbuf, stage,
             send_sems, recv_sems, copy_sem, out_sem):
        my = lax.axis_index("i")
        right = lax.rem(my + 1, N_DEV)
        left = lax.rem(my + N_DEV - 1, N_DEV)

        def wcast(wb, carry):
            cp = pltpu.make_async_copy(
                w_hbm.at[pl.ds(wb * WB, WB), :], wstage, copy_sem)
            cp.start()
            cp.wait()
            wbuf[pl.ds(wb * WB, WB), :] = wstage[:, :].astype(jnp.bfloat16)
            return carry
        lax.fori_loop(0, K // WB, wcast, 0)

        barrier_sem = pltpu.get_barrier_semaphore()
        for nbr in (left, right):
            pl.semaphore_signal(barrier_sem, inc=1, device_id=(nbr,),
                                device_id_type=pl.DeviceIdType.MESH)
        pl.semaphore_wait(barrier_sem, 2)

        def partial(chunk_j, b, ncol):
            cp = pltpu.make_async_copy(
                x_hbm.at[pl.ds(chunk_j * M_CHUNK + b * MB, MB), :], xbuf,
                copy_sem)
            cp.start()
            cp.wait()
            return jnp.dot(xbuf[:, :].astype(jnp.bfloat16),
                           wbuf[:, ncol:ncol + NH],
                           preferred_element_type=jnp.float32)

        hop = 0
        for r in range(2):
            ncol = r * NH
            for h in range(3):
                send_slot = hop % N_SLOTS
                recv_slot = (hop + 1) % N_SLOTS
                j = lax.rem(my - 1 - h + 2 * N_DEV, N_DEV)

                def sub(b, carry, h=h, send_slot=send_slot, j=j, ncol=ncol):
                    part = partial(j, b, ncol)
                    rows = pl.ds(b * MB, MB)
                    if h == 0:
                        comm[send_slot, rows, :] = part.astype(jnp.bfloat16)
                    else:
                        acc = part + comm[send_slot, rows, :].astype(jnp.float32)
                        comm[send_slot, rows, :] = acc.astype(jnp.bfloat16)
                    return carry
                lax.fori_loop(0, NSUB, sub, 0)

                rdma = pltpu.make_async_remote_copy(
                    src_ref=comm.at[send_slot],
                    dst_ref=comm.at[recv_slot],
                    send_sem=send_sems.at[send_slot],
                    recv_sem=recv_sems.at[recv_slot],
                    device_id=(right,),
                    device_id_type=pl.DeviceIdType.MESH)
                rdma.start()
                rdma.wait()
                hop += 1

            fin_slot = hop % N_SLOTS

            def fin(b, carry, fin_slot=fin_slot, ncol=ncol):
                part = partial(my, b, ncol)
                rows = pl.ds(b * MB, MB)
                stage[:, :] = part + comm[fin_slot, rows, :].astype(jnp.float32)
                ocp = pltpu.make_async_copy(
                    stage, out_hbm.at[rows, pl.ds(ncol, NH)], out_sem)
                ocp.start()
                ocp.wait()
                return carry
            lax.fori_loop(0, NSUB, fin, 0)

    return pl.pallas_call(
        body,
        out_shape=jax.ShapeDtypeStruct((M_CHUNK, N), jnp.float32),
        in_specs=[
            pl.BlockSpec(memory_space=pl.ANY),
            pl.BlockSpec(memory_space=pl.ANY),
        ],
        out_specs=pl.BlockSpec(memory_space=pl.ANY),
        scratch_shapes=[
            pltpu.VMEM((N_SLOTS, M_CHUNK, NH), jnp.bfloat16),
            pltpu.VMEM((K, N), jnp.bfloat16),
            pltpu.VMEM((WB, N), jnp.float32),
            pltpu.VMEM((MB, K), jnp.float32),
            pltpu.VMEM((MB, NH), jnp.float32),
            pltpu.SemaphoreType.DMA((N_SLOTS,)),
            pltpu.SemaphoreType.DMA((N_SLOTS,)),
            pltpu.SemaphoreType.DMA,
            pltpu.SemaphoreType.DMA,
        ],
        compiler_params=pltpu.CompilerParams(
            collective_id=0, vmem_limit_bytes=63 * 1024 * 1024),
    )(x, w_mat)


# device time: 390654 ns/iter; 2.1372x vs baseline; 2.1372x over previous
import jax
import jax.numpy as jnp
from jax import lax
from jax.experimental import pallas as pl
from jax.experimental.pallas import tpu as pltpu

N_DEV = 4
M_CHUNK = 2048
K = 2048
N = 4096
NH = N // 2
LANE_ROWS = M_CHUNK // 2
MB = 256
NSUB = LANE_ROWS // MB
WB = 256


def kernel(x, w_mat):
    assert x.shape == (N_DEV * M_CHUNK, K), x.shape
    assert w_mat.shape == (K, N), w_mat.shape

    def body(x_hbm, w_hbm, out_hbm, comm_cw, comm_ccw, wbuf, xbuf, stage,
             send_cw, recv_cw, send_ccw, recv_ccw, credits, copy_sem, out_sem):
        my = lax.axis_index("i")
        right = lax.rem(my + 1, N_DEV)
        left = lax.rem(my + N_DEV - 1, N_DEV)

        def wcast(wb, carry):
            cp = pltpu.make_async_copy(
                w_hbm.at[pl.ds(wb * WB, WB), :], stage, copy_sem)
            cp.start()
            cp.wait()
            wbuf[pl.ds(wb * WB, WB), :] = stage[:, :].astype(jnp.bfloat16)
            return carry
        lax.fori_loop(0, K // WB, wcast, 0)

        barrier_sem = pltpu.get_barrier_semaphore()
        for nbr in (left, right):
            pl.semaphore_signal(barrier_sem, inc=1, device_id=(nbr,),
                                device_id_type=pl.DeviceIdType.MESH)
        pl.semaphore_wait(barrier_sem, 2)

        dirs = (
            (comm_cw, send_cw, recv_cw, right, left, 0),
            (comm_ccw, send_ccw, recv_ccw, left, right, NH),
        )

        def chunk_of(di, h):
            if di == 0:
                return lax.rem(my - 1 - h + 2 * N_DEV, N_DEV)
            return lax.rem(my + 1 + h, N_DEV)

        def build(comm, lane, slot, chunk_j, ncol, add):
            def sub(b, carry):
                row0 = chunk_j * M_CHUNK + lane * LANE_ROWS + b * MB
                cp = pltpu.make_async_copy(
                    x_hbm.at[pl.ds(row0, MB), :], xbuf, copy_sem)
                cp.start()
                cp.wait()
                part = jnp.dot(xbuf[:, :].astype(jnp.bfloat16),
                               wbuf[:, ncol:ncol + NH],
                               preferred_element_type=jnp.float32)
                rows = pl.ds(b * MB, MB)
                if add:
                    part = part + comm[lane, slot, rows, :].astype(jnp.float32)
                comm[lane, slot, rows, :] = part.astype(jnp.bfloat16)
                return carry
            lax.fori_loop(0, NSUB, sub, 0)

        def make_rdma(comm, lane, sslot, rslot, ssem, rsem, dev):
            return pltpu.make_async_remote_copy(
                src_ref=comm.at[lane, sslot],
                dst_ref=comm.at[lane, rslot],
                send_sem=ssem.at[lane * 2 + sslot],
                recv_sem=rsem.at[lane * 2 + rslot],
                device_id=(dev,),
                device_id_type=pl.DeviceIdType.MESH)

        def send_credit(di, lane, upstream):
            pl.semaphore_signal(credits.at[di * 2 + lane], inc=1,
                                device_id=(upstream,),
                                device_id_type=pl.DeviceIdType.MESH)

        def wait_credit(di, lane):
            pl.semaphore_wait(credits.at[di * 2 + lane], 1)

        inflight = {}

        for lane in (0, 1):
            for di, (comm, ssem, rsem, dn, up, ncol) in enumerate(dirs):
                build(comm, lane, 0, chunk_of(di, 0), ncol, add=False)
                send_credit(di, lane, up)
                wait_credit(di, lane)
                r = make_rdma(comm, lane, 0, 1, ssem, rsem, dn)
                r.start()
                inflight[(di, lane, 0)] = r

        for h in (1, 2):
            s, rcv = h % 2, (h + 1) % 2
            for lane in (0, 1):
                for di, (comm, ssem, rsem, dn, up, ncol) in enumerate(dirs):
                    prev = inflight[(di, lane, h - 1)]
                    prev.wait_recv()
                    build(comm, lane, s, chunk_of(di, h), ncol, add=True)
                    prev.wait_send()
                    send_credit(di, lane, up)
                    wait_credit(di, lane)
                    r = make_rdma(comm, lane, s, rcv, ssem, rsem, dn)
                    r.start()
                    inflight[(di, lane, h)] = r

        for lane in (0, 1):
            for di, (comm, ssem, rsem, dn, up, ncol) in enumerate(dirs):
                inflight[(di, lane, 2)].wait_send()
                build(comm, lane, 0, my, ncol, add=False)

        for lane in (0, 1):
            for di, (comm, ssem, rsem, dn, up, ncol) in enumerate(dirs):
                inflight[(di, lane, 2)].wait_recv()

                def fin(b, carry, comm=comm, lane=lane, ncol=ncol):
                    rows = pl.ds(b * MB, MB)
                    acc = (comm[lane, 0, rows, :].astype(jnp.float32)
                           + comm[lane, 1, rows, :].astype(jnp.float32))
                    stage[:, ncol:ncol + NH] = acc
                    ocp = pltpu.make_async_copy(
                        stage.at[:, pl.ds(ncol, NH)],
                        out_hbm.at[pl.ds(lane * LANE_ROWS + b * MB, MB),
                                   pl.ds(ncol, NH)],
                        out_sem)
                    ocp.start()
                    ocp.wait()
                    return carry
                lax.fori_loop(0, NSUB, fin, 0)

    return pl.pallas_call(
        body,
        out_shape=jax.ShapeDtypeStruct((M_CHUNK, N), jnp.float32),
        in_specs=[
            pl.BlockSpec(memory_space=pl.ANY),
            pl.BlockSpec(memory_space=pl.ANY),
        ],
        out_specs=pl.BlockSpec(memory_space=pl.ANY),
        scratch_shapes=[
            pltpu.VMEM((2, 2, LANE_ROWS, NH), jnp.bfloat16),
            pltpu.VMEM((2, 2, LANE_ROWS, NH), jnp.bfloat16),
            pltpu.VMEM((K, N), jnp.bfloat16),
            pltpu.VMEM((MB, K), jnp.float32),
            pltpu.VMEM((WB, N), jnp.float32),
            pltpu.SemaphoreType.DMA((4,)),
            pltpu.SemaphoreType.DMA((4,)),
            pltpu.SemaphoreType.DMA((4,)),
            pltpu.SemaphoreType.DMA((4,)),
            pltpu.SemaphoreType.REGULAR((4,)),
            pltpu.SemaphoreType.DMA,
            pltpu.SemaphoreType.DMA,
        ],
        compiler_params=pltpu.CompilerParams(
            collective_id=0, vmem_limit_bytes=63 * 1024 * 1024),
    )(x, w_mat)


# device time: 362149 ns/iter; 2.3055x vs baseline; 1.0787x over previous
import jax
import jax.numpy as jnp
from jax import lax
from jax.experimental import pallas as pl
from jax.experimental.pallas import tpu as pltpu

N_DEV = 4
M_CHUNK = 2048
K = 2048
N = 4096
NH = N // 2
LANE_ROWS = M_CHUNK // 2
MB = 256
NSUB = LANE_ROWS // MB
WB = 256


def kernel(x, w_mat):
    assert x.shape == (N_DEV * M_CHUNK, K), x.shape
    assert w_mat.shape == (K, N), w_mat.shape

    def body(x_hbm, w_hbm, out_hbm, comm_cw, comm_ccw, wbuf, xbuf, stage,
             send_cw, recv_cw, send_ccw, recv_ccw, credits, copy_sem, out_sem):
        my = lax.axis_index("i")
        right = lax.rem(my + 1, N_DEV)
        left = lax.rem(my + N_DEV - 1, N_DEV)

        NWC = 2 * (K // WB)

        def wcp(i, slot):
            wb, half = divmod(i, 2)
            return pltpu.make_async_copy(
                w_hbm.at[pl.ds(wb * WB, WB), pl.ds(half * NH, NH)],
                stage.at[slot], out_sem.at[slot])
        wcp(0, 0).start()
        for i in range(NWC):
            s_ = i % 2
            if i + 1 < NWC:
                wcp(i + 1, (i + 1) % 2).start()
            wcp(i, s_).wait()
            wb, half = divmod(i, 2)
            wbuf[pl.ds(wb * WB, WB),
                 half * NH:half * NH + NH] = stage[s_, :, :].astype(jnp.bfloat16)

        barrier_sem = pltpu.get_barrier_semaphore()
        for nbr in (left, right):
            pl.semaphore_signal(barrier_sem, inc=1, device_id=(nbr,),
                                device_id_type=pl.DeviceIdType.MESH)
        pl.semaphore_wait(barrier_sem, 2)

        dirs = (
            (comm_cw, send_cw, recv_cw, right, left, 0),
            (comm_ccw, send_ccw, recv_ccw, left, right, NH),
        )

        def chunk_of(di, h):
            if di == 0:
                return lax.rem(my - 1 - h + 2 * N_DEV, N_DEV)
            return lax.rem(my + 1 + h, N_DEV)

        def build(comm, lane, slot, chunk_j, ncol, add):
            def sub(b, carry):
                row0 = chunk_j * M_CHUNK + lane * LANE_ROWS + b * MB
                cp = pltpu.make_async_copy(
                    x_hbm.at[pl.ds(row0, MB), :], xbuf, copy_sem)
                cp.start()
                cp.wait()
                part = jnp.dot(xbuf[:, :].astype(jnp.bfloat16),
                               wbuf[:, ncol:ncol + NH],
                               preferred_element_type=jnp.float32)
                rows = pl.ds(b * MB, MB)
                if add:
                    part = part + comm[lane, slot, rows, :].astype(jnp.float32)
                comm[lane, slot, rows, :] = part.astype(jnp.bfloat16)
                return carry
            lax.fori_loop(0, NSUB, sub, 0)

        def make_rdma(comm, lane, sslot, rslot, ssem, rsem, dev):
            return pltpu.make_async_remote_copy(
                src_ref=comm.at[lane, sslot],
                dst_ref=comm.at[lane, rslot],
                send_sem=ssem.at[lane * 2 + sslot],
                recv_sem=rsem.at[lane * 2 + rslot],
                device_id=(dev,),
                device_id_type=pl.DeviceIdType.MESH)

        def send_credit(di, lane, upstream):
            pl.semaphore_signal(credits.at[di * 2 + lane], inc=1,
                                device_id=(upstream,),
                                device_id_type=pl.DeviceIdType.MESH)

        def wait_credit(di, lane):
            pl.semaphore_wait(credits.at[di * 2 + lane], 1)

        inflight = {}

        for lane in (0, 1):
            for di, (comm, ssem, rsem, dn, up, ncol) in enumerate(dirs):
                build(comm, lane, 0, chunk_of(di, 0), ncol, add=False)
                send_credit(di, lane, up)
                wait_credit(di, lane)
                r = make_rdma(comm, lane, 0, 1, ssem, rsem, dn)
                r.start()
                inflight[(di, lane, 0)] = r

        for h in (1, 2):
            s, rcv = h % 2, (h + 1) % 2
            for lane in (0, 1):
                for di, (comm, ssem, rsem, dn, up, ncol) in enumerate(dirs):
                    prev = inflight[(di, lane, h - 1)]
                    prev.wait_recv()
                    build(comm, lane, s, chunk_of(di, h), ncol, add=True)
                    prev.wait_send()
                    send_credit(di, lane, up)
                    wait_credit(di, lane)
                    r = make_rdma(comm, lane, s, rcv, ssem, rsem, dn)
                    r.start()
                    inflight[(di, lane, h)] = r

        fin_i = 0
        pending = [None, None]
        for lane in (0, 1):
            for di, (comm, ssem, rsem, dn, up, ncol) in enumerate(dirs):
                r2 = inflight[(di, lane, 2)]
                for b in range(NSUB):
                    row0 = my * M_CHUNK + lane * LANE_ROWS + b * MB
                    cp = pltpu.make_async_copy(
                        x_hbm.at[pl.ds(row0, MB), :], xbuf, copy_sem)
                    cp.start()
                    cp.wait()
                    part = jnp.dot(xbuf[:, :].astype(jnp.bfloat16),
                                   wbuf[:, ncol:ncol + NH],
                                   preferred_element_type=jnp.float32)
                    if b == 0:
                        r2.wait_recv()
                    rows = pl.ds(b * MB, MB)
                    sslot = fin_i % 2
                    if pending[sslot] is not None:
                        pending[sslot].wait()
                    acc = part + comm[lane, 1, rows, :].astype(jnp.float32)
                    stage[sslot, :, :] = acc
                    ocp = pltpu.make_async_copy(
                        stage.at[sslot],
                        out_hbm.at[pl.ds(lane * LANE_ROWS + b * MB, MB),
                                   pl.ds(ncol, NH)],
                        out_sem.at[sslot])
                    ocp.start()
                    pending[sslot] = ocp
                    fin_i += 1
        for p in pending:
            if p is not None:
                p.wait()
        for lane in (0, 1):
            for di in (0, 1):
                inflight[(di, lane, 2)].wait_send()

    return pl.pallas_call(
        body,
        out_shape=jax.ShapeDtypeStruct((M_CHUNK, N), jnp.float32),
        in_specs=[
            pl.BlockSpec(memory_space=pl.ANY),
            pl.BlockSpec(memory_space=pl.ANY),
        ],
        out_specs=pl.BlockSpec(memory_space=pl.ANY),
        scratch_shapes=[
            pltpu.VMEM((2, 2, LANE_ROWS, NH), jnp.bfloat16),
            pltpu.VMEM((2, 2, LANE_ROWS, NH), jnp.bfloat16),
            pltpu.VMEM((K, N), jnp.bfloat16),
            pltpu.VMEM((MB, K), jnp.float32),
            pltpu.VMEM((2, WB, NH), jnp.float32),
            pltpu.SemaphoreType.DMA((4,)),
            pltpu.SemaphoreType.DMA((4,)),
            pltpu.SemaphoreType.DMA((4,)),
            pltpu.SemaphoreType.DMA((4,)),
            pltpu.SemaphoreType.REGULAR((4,)),
            pltpu.SemaphoreType.DMA,
            pltpu.SemaphoreType.DMA((2,)),
        ],
        compiler_params=pltpu.CompilerParams(
            collective_id=0, vmem_limit_bytes=66_998_000),
    )(x, w_mat)


# device time: 361046 ns/iter; 2.3125x vs baseline; 1.0031x over previous
import jax
import jax.numpy as jnp
from jax import lax
from jax.experimental import pallas as pl
from jax.experimental.pallas import tpu as pltpu

N_DEV = 4
M_CHUNK = 2048
K = 2048
N = 4096
NH = N // 2
LANE_ROWS = M_CHUNK // 2
MB = 256
NSUB = LANE_ROWS // MB
WB = 256


def kernel(x, w_mat):
    assert x.shape == (N_DEV * M_CHUNK, K), x.shape
    assert w_mat.shape == (K, N), w_mat.shape

    def body(x_hbm, w_hbm, out_hbm, comm_cw, comm_ccw, wbuf, xbuf, stage,
             send_cw, recv_cw, send_ccw, recv_ccw, credits, copy_sem, out_sem):
        my = lax.axis_index("i")
        right = lax.rem(my + 1, N_DEV)
        left = lax.rem(my + N_DEV - 1, N_DEV)

        def wcast(half):
            def wcp(wb, slot):
                return pltpu.make_async_copy(
                    w_hbm.at[pl.ds(wb * WB, WB), pl.ds(half * NH, NH)],
                    stage.at[slot], out_sem.at[slot])
            nw = K // WB
            wcp(0, 0).start()
            for wb in range(nw):
                s_ = wb % 2
                if wb + 1 < nw:
                    wcp(wb + 1, (wb + 1) % 2).start()
                wcp(wb, s_).wait()
                wbuf[pl.ds(wb * WB, WB),
                     half * NH:half * NH + NH] = (
                         stage[s_, :, :].astype(jnp.bfloat16))

        barrier_sem = pltpu.get_barrier_semaphore()
        for nbr in (left, right):
            pl.semaphore_signal(barrier_sem, inc=1, device_id=(nbr,),
                                device_id_type=pl.DeviceIdType.MESH)
        pl.semaphore_wait(barrier_sem, 2)

        dirs = (
            (comm_cw, send_cw, recv_cw, right, left, 0),
            (comm_ccw, send_ccw, recv_ccw, left, right, NH),
        )

        def chunk_of(di, h):
            if di == 0:
                return lax.rem(my - 1 - h + 2 * N_DEV, N_DEV)
            return lax.rem(my + 1 + h, N_DEV)

        def build(comm, lane, slot, chunk_j, ncol, add):
            def sub(b, carry):
                row0 = chunk_j * M_CHUNK + lane * LANE_ROWS + b * MB
                cp = pltpu.make_async_copy(
                    x_hbm.at[pl.ds(row0, MB), :], xbuf, copy_sem)
                cp.start()
                cp.wait()
                part = jnp.dot(xbuf[:, :].astype(jnp.bfloat16),
                               wbuf[:, ncol:ncol + NH],
                               preferred_element_type=jnp.float32)
                rows = pl.ds(b * MB, MB)
                if add:
                    part = part + comm[lane, slot, rows, :].astype(jnp.float32)
                comm[lane, slot, rows, :] = part.astype(jnp.bfloat16)
                return carry
            lax.fori_loop(0, NSUB, sub, 0)

        def make_rdma(comm, lane, sslot, rslot, ssem, rsem, dev):
            return pltpu.make_async_remote_copy(
                src_ref=comm.at[lane, sslot],
                dst_ref=comm.at[lane, rslot],
                send_sem=ssem.at[lane * 2 + sslot],
                recv_sem=rsem.at[lane * 2 + rslot],
                device_id=(dev,),
                device_id_type=pl.DeviceIdType.MESH)

        def send_credit(di, lane, upstream):
            pl.semaphore_signal(credits.at[di * 2 + lane], inc=1,
                                device_id=(upstream,),
                                device_id_type=pl.DeviceIdType.MESH)

        def wait_credit(di, lane):
            pl.semaphore_wait(credits.at[di * 2 + lane], 1)

        inflight = {}

        def hop0_send(lane, di):
            comm, ssem, rsem, dn, up, ncol = dirs[di]
            build(comm, lane, 0, chunk_of(di, 0), ncol, add=False)
            send_credit(di, lane, up)
            wait_credit(di, lane)
            r = make_rdma(comm, lane, 0, 1, ssem, rsem, dn)
            r.start()
            inflight[(di, lane, 0)] = r

        wcast(0)
        hop0_send(0, 0)
        wcast(1)
        hop0_send(0, 1)
        hop0_send(1, 0)
        hop0_send(1, 1)

        for h in (1, 2):
            s, rcv = h % 2, (h + 1) % 2
            for lane in (0, 1):
                for di, (comm, ssem, rsem, dn, up, ncol) in enumerate(dirs):
                    prev = inflight[(di, lane, h - 1)]
                    prev.wait_recv()
                    build(comm, lane, s, chunk_of(di, h), ncol, add=True)
                    prev.wait_send()
                    send_credit(di, lane, up)
                    wait_credit(di, lane)
                    r = make_rdma(comm, lane, s, rcv, ssem, rsem, dn)
                    r.start()
                    inflight[(di, lane, h)] = r

        fin_i = 0
        pending = [None, None]
        for lane in (0, 1):
            for di, (comm, ssem, rsem, dn, up, ncol) in enumerate(dirs):
                r2 = inflight[(di, lane, 2)]
                for b in range(NSUB):
                    row0 = my * M_CHUNK + lane * LANE_ROWS + b * MB
                    cp = pltpu.make_async_copy(
                        x_hbm.at[pl.ds(row0, MB), :], xbuf, copy_sem)
                    cp.start()
                    cp.wait()
                    part = jnp.dot(xbuf[:, :].astype(jnp.bfloat16),
                                   wbuf[:, ncol:ncol + NH],
                                   preferred_element_type=jnp.float32)
                    if b == 0:
                        r2.wait_recv()
                    rows = pl.ds(b * MB, MB)
                    sslot = fin_i % 2
                    if pending[sslot] is not None:
                        pending[sslot].wait()
                    acc = part + comm[lane, 1, rows, :].astype(jnp.float32)
                    stage[sslot, :, :] = acc
                    ocp = pltpu.make_async_copy(
                        stage.at[sslot],
                        out_hbm.at[pl.ds(lane * LANE_ROWS + b * MB, MB),
                                   pl.ds(ncol, NH)],
                        out_sem.at[sslot])
                    ocp.start()
                    pending[sslot] = ocp
                    fin_i += 1
        for p in pending:
            if p is not None:
                p.wait()
        for lane in (0, 1):
            for di in (0, 1):
                inflight[(di, lane, 2)].wait_send()

    return pl.pallas_call(
        body,
        out_shape=jax.ShapeDtypeStruct((M_CHUNK, N), jnp.float32),
        in_specs=[
            pl.BlockSpec(memory_space=pl.ANY),
            pl.BlockSpec(memory_space=pl.ANY),
        ],
        out_specs=pl.BlockSpec(memory_space=pl.ANY),
        scratch_shapes=[
            pltpu.VMEM((2, 2, LANE_ROWS, NH), jnp.bfloat16),
            pltpu.VMEM((2, 2, LANE_ROWS, NH), jnp.bfloat16),
            pltpu.VMEM((K, N), jnp.bfloat16),
            pltpu.VMEM((MB, K), jnp.float32),
            pltpu.VMEM((2, WB, NH), jnp.float32),
            pltpu.SemaphoreType.DMA((4,)),
            pltpu.SemaphoreType.DMA((4,)),
            pltpu.SemaphoreType.DMA((4,)),
            pltpu.SemaphoreType.DMA((4,)),
            pltpu.SemaphoreType.REGULAR((4,)),
            pltpu.SemaphoreType.DMA,
            pltpu.SemaphoreType.DMA((2,)),
        ],
        compiler_params=pltpu.CompilerParams(
            collective_id=0, vmem_limit_bytes=66_998_000),
    )(x, w_mat)
